# baseline (speedup 1.0000x reference)
import functools

import jax
import jax.numpy as jnp
from jax import lax
from jax.experimental import pallas as pl
from jax.experimental.pallas import tpu as pltpu
from jax.experimental.pallas import tpu_sc as plsc

N = 10000
E = 320000
D_IN = 128
H = 64
L = 4

W = 128
NC = 2
NS = 16
NW = NC * NS
CB = 128
CH = 81
NBUF = 3
ET = CH * CB
EP = NW * ET
NP = 10240
RPT = NP // NS

BE = 2048
BN = 1024

def _silu(x):
    return x * jax.nn.sigmoid(x)


def _gather_body(ts_hbm, tr_hbm, sg_hbm, rg_hbm, es_hbm, er_hbm,
                 sidx, ridx, bs0, bs1, br0, br1, sg0, sg1, sh0, sh1):
    buf_s = (bs0, bs1)
    buf_r = (br0, br1)
    sem_s = (sg0, sg1)
    sem_r = (sh0, sh1)
    c = lax.axis_index("c")
    t = lax.axis_index("s")
    wid = t * NC + c
    pltpu.sync_copy(sg_hbm.at[wid], sidx)
    pltpu.sync_copy(rg_hbm.at[wid], ridx)
    ebase = wid * ET

    def body(j, carry):
        cs = pltpu.async_copy(ts_hbm.at[sidx.at[j]], buf_s[0], sem_s[0])
        cr = pltpu.async_copy(tr_hbm.at[ridx.at[j]], buf_r[0], sem_r[0])
        cs.wait()
        cr.wait()
        row0 = ebase + j * CB
        pltpu.sync_copy(buf_s[0], es_hbm.at[pl.ds(row0, CB)])
        pltpu.sync_copy(buf_r[0], er_hbm.at[pl.ds(row0, CB)])
        return carry

    lax.fori_loop(0, CH, body, 0)


def _scatter_body(eo_hbm, rs_hbm, p0_hbm, p1_hbm,
                  ridx, eb0, eb1, acc, sr0, sr1):
    eb = (eb0, eb1)
    sem_r = (sr0, sr1)
    c = lax.axis_index("c")
    t = lax.axis_index("s")
    wid = t * NC + c
    ebase = wid * ET

    del eb, sem_r
    pltpu.sync_copy(rs_hbm.at[wid], ridx)

    def zrow(i, carry):
        for k in range(W // 16):
            eb0[i, pl.ds(k * 16, 16)] = jnp.zeros((16,), jnp.float32)
        return carry
    lax.fori_loop(0, CB, zrow, 0)
    for q in range(RPT // CB):
        pltpu.sync_copy(eb0, acc.at[pl.ds(t * RPT + q * CB, CB)])
    plsc.subcore_barrier()

    def body(j, carry):
        pltpu.sync_copy(eo_hbm.at[pl.ds(ebase + j * CB, CB)], eb0)
        pltpu.sync_copy(eb0, acc.at[ridx.at[j]], add=True)
        return carry

    lax.fori_loop(0, CH, body, 0)
    plsc.subcore_barrier()

    for q in range(RPT // CB):
        row0 = t * RPT + q * CB
        pltpu.sync_copy(acc.at[pl.ds(row0, CB)], eb0)

        @pl.when(c == 0)
        def _():
            pltpu.sync_copy(eb0, p0_hbm.at[pl.ds(row0, CB)])

        @pl.when(c == 1)
        def _():
            pltpu.sync_copy(eb0, p1_hbm.at[pl.ds(row0, CB)])


@functools.lru_cache(maxsize=1)
def _sc_kernels():
    mesh = plsc.VectorSubcoreMesh(
        core_axis_name="c", subcore_axis_name="s",
        num_cores=NC, num_subcores=NS)
    gather = pl.kernel(
        _gather_body,
        out_type=(jax.ShapeDtypeStruct((EP, W), jnp.float32),
                  jax.ShapeDtypeStruct((EP, W), jnp.float32)),
        mesh=mesh,
        scratch_types=(
            [pltpu.VMEM((CH, CB), jnp.int32)] * 2
            + [pltpu.VMEM((CB, W), jnp.float32)] * 4
            + [pltpu.SemaphoreType.DMA] * 4
        ),
    )
    scatter = pl.kernel(
        _scatter_body,
        out_type=(jax.ShapeDtypeStruct((NP, W), jnp.float32),
                  jax.ShapeDtypeStruct((NP, W), jnp.float32)),
        mesh=mesh,
        scratch_types=(
            [pltpu.VMEM((CH, CB), jnp.int32)]
            + [pltpu.VMEM((CB, W), jnp.float32)] * 2
            + [pltpu.VMEM_SHARED((NP, W), jnp.float32)]
            + [pltpu.SemaphoreType.DMA] * 2
        ),
    )
    return gather, scatter


def _embed_body(x_ref, pos_ref, win_ref, bin_ref, h_ref, ts_ref, tr_ref):
    xb = x_ref[...]
    pb = pos_ref[...]
    h0 = xb @ win_ref[...] + bin_ref[...]
    z = jnp.zeros((xb.shape[0], W - H - 4), jnp.float32)
    h_ref[...] = h0
    ts_ref[...] = jnp.concatenate([h0, pb, z], axis=1)
    tr_ref[...] = jnp.concatenate([h0, -pb, z], axis=1)


def _edge_body(es_ref, er_ref, we1_ref, be1_ref, we2_ref, be2_ref,
               wp1_ref, bp1_ref, wp2_ref, bp2_ref, eo_ref):
    es = es_ref[...]
    er = er_ref[...]
    d16 = es[:, H:W] + er[:, H:W]
    dsq = d16 * d16
    radial = dsq[:, 0:1] + dsq[:, 1:2] + dsq[:, 2:3]
    cc = jnp.concatenate([es[:, :H], er[:, :H], radial], axis=1)
    m = _silu(cc @ we1_ref[...] + be1_ref[...])
    m = _silu(m @ we2_ref[...] + be2_ref[...])
    q1 = _silu(m @ wp1_ref[...] + bp1_ref[...])
    pw = q1 @ wp2_ref[...] + bp2_ref[...]
    t16 = jnp.clip(d16 * pw, -100.0, 100.0)
    eo_ref[...] = jnp.concatenate([m, t16], axis=1)


def _node_body(h_ref, pos_ref, p0_ref, p1_ref, wn1_ref, bn1_ref,
               wn2_ref, bn2_ref, hn_ref, posn_ref, ts_ref, tr_ref):
    h = h_ref[...]
    p = p0_ref[...] + p1_ref[...]
    u1 = _silu(jnp.concatenate([h, p[:, :H]], axis=1) @ wn1_ref[...]
               + bn1_ref[...])
    hn = h + (u1 @ wn2_ref[...] + bn2_ref[...])
    posn = pos_ref[...] + p[:, H:H + 4]
    z = jnp.zeros((h.shape[0], W - H - 4), jnp.float32)
    hn_ref[...] = hn
    posn_ref[...] = posn
    ts_ref[...] = jnp.concatenate([hn, posn, z], axis=1)
    tr_ref[...] = jnp.concatenate([hn, -posn, z], axis=1)


def _node_last_body(h_ref, pos_ref, p0_ref, p1_ref, wn1_ref, bn1_ref,
                    wn2_ref, bn2_ref, wout_ref, bout_ref,
                    out_ref, posn_ref):
    h = h_ref[...]
    p = p0_ref[...] + p1_ref[...]
    u1 = _silu(jnp.concatenate([h, p[:, :H]], axis=1) @ wn1_ref[...]
               + bn1_ref[...])
    hn = h + (u1 @ wn2_ref[...] + bn2_ref[...])
    out_ref[...] = hn @ wout_ref[...] + bout_ref[...]
    posn_ref[...] = pos_ref[...] + p[:, H:H + 4]


def _full(shape):
    return pl.BlockSpec(shape, lambda i: (0,) * len(shape))


def _rows(b, w):
    return pl.BlockSpec((b, w), lambda i: (i, 0))


def _tc_embed(xp, posp, w_in, b_in):
    return pl.pallas_call(
        _embed_body,
        grid=(NP // BN,),
        in_specs=[_rows(BN, D_IN), _rows(BN, 4), _full((D_IN, H)),
                  _full((1, H))],
        out_specs=[_rows(BN, H), _rows(BN, W), _rows(BN, W)],
        out_shape=[jax.ShapeDtypeStruct((NP, H), jnp.float32),
                   jax.ShapeDtypeStruct((NP, W), jnp.float32),
                   jax.ShapeDtypeStruct((NP, W), jnp.float32)],
    )(xp, posp, w_in, b_in)


def _tc_edge(es, er, we1, be1, we2, be2, wp1, bp1, wp2, bp2):
    return pl.pallas_call(
        _edge_body,
        grid=(EP // BE,),
        in_specs=[_rows(BE, W), _rows(BE, W), _full((2 * H + 1, H)),
                  _full((1, H)), _full((H, H)), _full((1, H)), _full((H, H)),
                  _full((1, H)), _full((H, 1)), _full((1, 1))],
        out_specs=[_rows(BE, W)],
        out_shape=[jax.ShapeDtypeStruct((EP, W), jnp.float32)],
    )(es, er, we1, be1, we2, be2, wp1, bp1, wp2, bp2)[0]


def _tc_node(h, posp, p0, p1, wn1, bn1, wn2, bn2):
    return pl.pallas_call(
        _node_body,
        grid=(NP // BN,),
        in_specs=[_rows(BN, H), _rows(BN, 4), _rows(BN, W), _rows(BN, W),
                  _full((2 * H, H)), _full((1, H)), _full((H, H)),
                  _full((1, H))],
        out_specs=[_rows(BN, H), _rows(BN, 4), _rows(BN, W), _rows(BN, W)],
        out_shape=[jax.ShapeDtypeStruct((NP, H), jnp.float32),
                   jax.ShapeDtypeStruct((NP, 4), jnp.float32),
                   jax.ShapeDtypeStruct((NP, W), jnp.float32),
                   jax.ShapeDtypeStruct((NP, W), jnp.float32)],
    )(h, posp, p0, p1, wn1, bn1, wn2, bn2)


def _tc_node_last(h, posp, p0, p1, wn1, bn1, wn2, bn2, w_out, b_out):
    return pl.pallas_call(
        _node_last_body,
        grid=(NP // BN,),
        in_specs=[_rows(BN, H), _rows(BN, 4), _rows(BN, W), _rows(BN, W),
                  _full((2 * H, H)), _full((1, H)), _full((H, H)),
                  _full((1, H)), _full((H, H)), _full((1, H))],
        out_specs=[_rows(BN, H), _rows(BN, 4)],
        out_shape=[jax.ShapeDtypeStruct((NP, H), jnp.float32),
                   jax.ShapeDtypeStruct((NP, 4), jnp.float32)],
    )(h, posp, p0, p1, wn1, bn1, wn2, bn2, w_out, b_out)


def kernel(x, pos, edge_index, W_in, b_in, We1, be1, We2, be2, Wp1, bp1,
           Wp2, bp2, Wn1, bn1, Wn2, bn2, W_out, b_out):
    f32 = jnp.float32
    s = edge_index[0].astype(jnp.int32)
    r = edge_index[1].astype(jnp.int32)
    padg = jnp.zeros((EP - E,), jnp.int32)
    sg = jnp.concatenate([s, padg]).reshape(NW, CH, CB)
    rg = jnp.concatenate([r, padg]).reshape(NW, CH, CB)
    rs = jnp.concatenate(
        [r, jnp.full((EP - E,), NP - 1, jnp.int32)]).reshape(NW, CH, CB)

    xp = jnp.zeros((NP, D_IN), f32).at[:N].set(x)
    posp = jnp.zeros((NP, 4), f32).at[:N, :3].set(pos)
    r2 = lambda v: v.reshape(1, -1)

    sc_gather, sc_scatter = _sc_kernels()
    h, ts, tr = _tc_embed(xp, posp, W_in, r2(b_in))
    for l in range(L):
        es, er = sc_gather(ts, tr, sg, rg)
        eo = _tc_edge(es, er, We1[l], r2(be1[l]), We2[l], r2(be2[l]),
                      Wp1[l], r2(bp1[l]), Wp2[l], r2(bp2[l]))
        p0, p1 = sc_scatter(eo, rs)
        if l < L - 1:
            h, posp, ts, tr = _tc_node(
                h, posp, p0, p1, Wn1[l], r2(bn1[l]), Wn2[l], r2(bn2[l]))
        else:
            out, posp = _tc_node_last(
                h, posp, p0, p1, Wn1[l], r2(bn1[l]), Wn2[l], r2(bn2[l]),
                W_out, r2(b_out))
    return (out[:N], posp[:N, :3])


# speedup vs baseline: 1.4969x; 1.4969x over previous
import functools

import jax
import jax.numpy as jnp
from jax import lax
from jax.experimental import pallas as pl
from jax.experimental.pallas import tpu as pltpu
from jax.experimental.pallas import tpu_sc as plsc

N = 10000
E = 320000
D_IN = 128
H = 64
L = 4

W = 128
NC = 2
NS = 16
NW = NC * NS
CB = 128
CH = 79
NBUF = 3
ET = CH * CB
EP = NW * ET
NP = 10240
RPT = NP // NS

BE = 2048
BN = 1024

def _silu(x):
    return x * jax.nn.sigmoid(x)


def _gather_body(ts_hbm, tr_hbm, sg_hbm, rg_hbm, es_hbm, er_hbm,
                 sidx, ridx, bs0, bs1, br0, br1, sg0, sg1, sh0, sh1):
    buf_s = (bs0, bs1)
    buf_r = (br0, br1)
    sem_s = (sg0, sg1)
    sem_r = (sh0, sh1)
    c = lax.axis_index("c")
    t = lax.axis_index("s")
    wid = t * NC + c
    pltpu.sync_copy(sg_hbm.at[wid], sidx)
    pltpu.sync_copy(rg_hbm.at[wid], ridx)
    ebase = wid * ET

    def body(j, carry):
        cs = pltpu.async_copy(ts_hbm.at[sidx.at[j]], buf_s[0], sem_s[0])
        cr = pltpu.async_copy(tr_hbm.at[ridx.at[j]], buf_r[0], sem_r[0])
        cs.wait()
        cr.wait()
        row0 = ebase + j * CB
        pltpu.sync_copy(buf_s[0], es_hbm.at[pl.ds(row0, CB)])
        pltpu.sync_copy(buf_r[0], er_hbm.at[pl.ds(row0, CB)])
        return carry

    lax.fori_loop(0, CH, body, 0)


def _scatter_body(eo_hbm, rs_hbm, p0_hbm, p1_hbm,
                  ridx, eb0, eb1, acc, sr0, sr1):
    eb = (eb0, eb1)
    sem_r = (sr0, sr1)
    c = lax.axis_index("c")
    t = lax.axis_index("s")
    wid = t * NC + c
    ebase = wid * ET

    del eb, sem_r
    pltpu.sync_copy(rs_hbm.at[wid], ridx)

    def zrow(i, carry):
        for k in range(W // 16):
            eb0[i, pl.ds(k * 16, 16)] = jnp.zeros((16,), jnp.float32)
        return carry
    lax.fori_loop(0, CB, zrow, 0)
    for q in range(RPT // CB):
        pltpu.sync_copy(eb0, acc.at[pl.ds(t * RPT + q * CB, CB)])
    plsc.subcore_barrier()

    def body(j, carry):
        pltpu.sync_copy(eo_hbm.at[pl.ds(ebase + j * CB, CB)], eb0)
        pltpu.sync_copy(eb0, acc.at[ridx.at[j]], add=True)
        return carry

    lax.fori_loop(0, CH, body, 0)
    plsc.subcore_barrier()

    for q in range(RPT // CB):
        row0 = t * RPT + q * CB
        pltpu.sync_copy(acc.at[pl.ds(row0, CB)], eb0)

        @pl.when(c == 0)
        def _():
            pltpu.sync_copy(eb0, p0_hbm.at[pl.ds(row0, CB)])

        @pl.when(c == 1)
        def _():
            pltpu.sync_copy(eb0, p1_hbm.at[pl.ds(row0, CB)])


@functools.lru_cache(maxsize=1)
def _sc_kernels():
    mesh = plsc.VectorSubcoreMesh(
        core_axis_name="c", subcore_axis_name="s",
        num_cores=NC, num_subcores=NS)
    gather = pl.kernel(
        _gather_body,
        out_type=(jax.ShapeDtypeStruct((EP, W), jnp.float32),
                  jax.ShapeDtypeStruct((EP, W), jnp.float32)),
        mesh=mesh,
        scratch_types=(
            [pltpu.VMEM((CH, CB), jnp.int32)] * 2
            + [pltpu.VMEM((CB, W), jnp.float32)] * 4
            + [pltpu.SemaphoreType.DMA] * 4
        ),
    )
    scatter = pl.kernel(
        _scatter_body,
        out_type=(jax.ShapeDtypeStruct((NP, W), jnp.float32),
                  jax.ShapeDtypeStruct((NP, W), jnp.float32)),
        mesh=mesh,
        scratch_types=(
            [pltpu.VMEM((CH, CB), jnp.int32)]
            + [pltpu.VMEM((CB, W), jnp.float32)] * 2
            + [pltpu.VMEM_SHARED((NP, W), jnp.float32)]
            + [pltpu.SemaphoreType.DMA] * 2
        ),
    )
    return gather, scatter


def _embed_body(x_ref, pos_ref, win_ref, bin_ref, h_ref, ts_ref, tr_ref):
    xb = x_ref[...]
    pb = pos_ref[...]
    h0 = xb @ win_ref[...] + bin_ref[...]
    z = jnp.zeros((xb.shape[0], W - H - 4), jnp.float32)
    h_ref[...] = h0
    ts_ref[...] = jnp.concatenate([h0, pb, z], axis=1)
    tr_ref[...] = jnp.concatenate([h0, -pb, z], axis=1)


def _edge_body(es_ref, er_ref, we1_ref, be1_ref, we2_ref, be2_ref,
               wp1_ref, bp1_ref, wp2_ref, bp2_ref, eo_ref):
    es = es_ref[...]
    er = er_ref[...]
    d16 = es[:, H:W] + er[:, H:W]
    dsq = d16 * d16
    radial = dsq[:, 0:1] + dsq[:, 1:2] + dsq[:, 2:3]
    cc = jnp.concatenate([es[:, :H], er[:, :H], radial], axis=1)
    m = _silu(cc @ we1_ref[...] + be1_ref[...])
    m = _silu(m @ we2_ref[...] + be2_ref[...])
    q1 = _silu(m @ wp1_ref[...] + bp1_ref[...])
    pw = q1 @ wp2_ref[...] + bp2_ref[...]
    t16 = jnp.clip(d16 * pw, -100.0, 100.0)
    eo_ref[...] = jnp.concatenate([m, t16], axis=1)


def _node_body(h_ref, pos_ref, p0_ref, p1_ref, wn1_ref, bn1_ref,
               wn2_ref, bn2_ref, hn_ref, posn_ref, ts_ref, tr_ref):
    h = h_ref[...]
    p = p0_ref[...] + p1_ref[...]
    u1 = _silu(jnp.concatenate([h, p[:, :H]], axis=1) @ wn1_ref[...]
               + bn1_ref[...])
    hn = h + (u1 @ wn2_ref[...] + bn2_ref[...])
    posn = pos_ref[...] + p[:, H:H + 4]
    z = jnp.zeros((h.shape[0], W - H - 4), jnp.float32)
    hn_ref[...] = hn
    posn_ref[...] = posn
    ts_ref[...] = jnp.concatenate([hn, posn, z], axis=1)
    tr_ref[...] = jnp.concatenate([hn, -posn, z], axis=1)


def _node_last_body(h_ref, pos_ref, p0_ref, p1_ref, wn1_ref, bn1_ref,
                    wn2_ref, bn2_ref, wout_ref, bout_ref,
                    out_ref, posn_ref):
    h = h_ref[...]
    p = p0_ref[...] + p1_ref[...]
    u1 = _silu(jnp.concatenate([h, p[:, :H]], axis=1) @ wn1_ref[...]
               + bn1_ref[...])
    hn = h + (u1 @ wn2_ref[...] + bn2_ref[...])
    out_ref[...] = hn @ wout_ref[...] + bout_ref[...]
    posn_ref[...] = pos_ref[...] + p[:, H:H + 4]


def _full(shape):
    return pl.BlockSpec(shape, lambda i: (0,) * len(shape))


def _rows(b, w):
    return pl.BlockSpec((b, w), lambda i: (i, 0))


def _tc_embed(xp, posp, w_in, b_in):
    return pl.pallas_call(
        _embed_body,
        grid=(NP // BN,),
        in_specs=[_rows(BN, D_IN), _rows(BN, 4), _full((D_IN, H)),
                  _full((1, H))],
        out_specs=[_rows(BN, H), _rows(BN, W), _rows(BN, W)],
        out_shape=[jax.ShapeDtypeStruct((NP, H), jnp.float32),
                   jax.ShapeDtypeStruct((NP, W), jnp.float32),
                   jax.ShapeDtypeStruct((NP, W), jnp.float32)],
    )(xp, posp, w_in, b_in)


def _tc_edge(es, er, we1, be1, we2, be2, wp1, bp1, wp2, bp2):
    return pl.pallas_call(
        _edge_body,
        grid=(EP // BE,),
        in_specs=[_rows(BE, W), _rows(BE, W), _full((2 * H + 1, H)),
                  _full((1, H)), _full((H, H)), _full((1, H)), _full((H, H)),
                  _full((1, H)), _full((H, 1)), _full((1, 1))],
        out_specs=[_rows(BE, W)],
        out_shape=[jax.ShapeDtypeStruct((EP, W), jnp.float32)],
    )(es, er, we1, be1, we2, be2, wp1, bp1, wp2, bp2)[0]


def _tc_node(h, posp, p0, p1, wn1, bn1, wn2, bn2):
    return pl.pallas_call(
        _node_body,
        grid=(NP // BN,),
        in_specs=[_rows(BN, H), _rows(BN, 4), _rows(BN, W), _rows(BN, W),
                  _full((2 * H, H)), _full((1, H)), _full((H, H)),
                  _full((1, H))],
        out_specs=[_rows(BN, H), _rows(BN, 4), _rows(BN, W), _rows(BN, W)],
        out_shape=[jax.ShapeDtypeStruct((NP, H), jnp.float32),
                   jax.ShapeDtypeStruct((NP, 4), jnp.float32),
                   jax.ShapeDtypeStruct((NP, W), jnp.float32),
                   jax.ShapeDtypeStruct((NP, W), jnp.float32)],
    )(h, posp, p0, p1, wn1, bn1, wn2, bn2)


def _tc_node_last(h, posp, p0, p1, wn1, bn1, wn2, bn2, w_out, b_out):
    return pl.pallas_call(
        _node_last_body,
        grid=(NP // BN,),
        in_specs=[_rows(BN, H), _rows(BN, 4), _rows(BN, W), _rows(BN, W),
                  _full((2 * H, H)), _full((1, H)), _full((H, H)),
                  _full((1, H)), _full((H, H)), _full((1, H))],
        out_specs=[_rows(BN, H), _rows(BN, 4)],
        out_shape=[jax.ShapeDtypeStruct((NP, H), jnp.float32),
                   jax.ShapeDtypeStruct((NP, 4), jnp.float32)],
    )(h, posp, p0, p1, wn1, bn1, wn2, bn2, w_out, b_out)


def kernel(x, pos, edge_index, W_in, b_in, We1, be1, We2, be2, Wp1, bp1,
           Wp2, bp2, Wn1, bn1, Wn2, bn2, W_out, b_out):
    f32 = jnp.float32
    s = edge_index[0].astype(jnp.int32)
    r = edge_index[1].astype(jnp.int32)
    padg = jnp.zeros((EP - E,), jnp.int32)
    sg = jnp.concatenate([s, padg]).reshape(NW, CH, CB)
    rg = jnp.concatenate([r, padg]).reshape(NW, CH, CB)
    rs = jnp.concatenate(
        [r, jnp.full((EP - E,), NP - 1, jnp.int32)]).reshape(NW, CH, CB)

    xp = jnp.zeros((NP, D_IN), f32).at[:N].set(x)
    posp = jnp.zeros((NP, 4), f32).at[:N, :3].set(pos)
    r2 = lambda v: v.reshape(1, -1)

    sc_gather, sc_scatter = _sc_kernels()
    h, ts, tr = _tc_embed(xp, posp, W_in, r2(b_in))
    for l in range(L):
        es, er = sc_gather(ts, tr, sg, rg)
        eo = _tc_edge(es, er, We1[l], r2(be1[l]), We2[l], r2(be2[l]),
                      Wp1[l], r2(bp1[l]), Wp2[l], r2(bp2[l]))
        p0, p1 = sc_scatter(eo, rs)
        if l < L - 1:
            h, posp, ts, tr = _tc_node(
                h, posp, p0, p1, Wn1[l], r2(bn1[l]), Wn2[l], r2(bn2[l]))
        else:
            out, posp = _tc_node_last(
                h, posp, p0, p1, Wn1[l], r2(bn1[l]), Wn2[l], r2(bn2[l]),
                W_out, r2(b_out))
    return (out[:N], posp[:N, :3])


# speedup vs baseline: 1.7234x; 1.1513x over previous
import functools

import jax
import jax.numpy as jnp
from jax import lax
from jax.experimental import pallas as pl
from jax.experimental.pallas import tpu as pltpu
from jax.experimental.pallas import tpu_sc as plsc

N = 10000
E = 320000
D_IN = 128
H = 64
L = 4

W = 128
NC = 2
NS = 16
NW = NC * NS
CB = 128
CH = 79
NBUF = 3
ET = CH * CB
EP = NW * ET
NP = 10240
RPT = NP // NS

BE = 2048
BN = 1024

def _silu(x):
    return x * jax.nn.sigmoid(x)


def _gather_body(ts_hbm, tr_hbm, sg_hbm, rg_hbm, es_hbm, er_hbm,
                 sidx, ridx, bs0, bs1, br0, br1, sg0, sg1, sh0, sh1):
    buf_s = (bs0, bs1)
    buf_r = (br0, br1)
    sem_s = (sg0, sg1)
    sem_r = (sh0, sh1)
    c = lax.axis_index("c")
    t = lax.axis_index("s")
    wid = t * NC + c
    pltpu.sync_copy(sg_hbm.at[wid], sidx)
    pltpu.sync_copy(rg_hbm.at[wid], ridx)
    ebase = wid * ET

    def start(j, b):
        pltpu.async_copy(ts_hbm.at[sidx.at[j]], buf_s[b], sem_s[b])
        pltpu.async_copy(tr_hbm.at[ridx.at[j]], buf_r[b], sem_r[b])

    def finish(j, b):
        pltpu.make_async_copy(ts_hbm.at[pl.ds(0, CB)], buf_s[b],
                              sem_s[b]).wait()
        pltpu.make_async_copy(tr_hbm.at[pl.ds(0, CB)], buf_r[b],
                              sem_r[b]).wait()
        row0 = ebase + j * CB
        pltpu.sync_copy(buf_s[b], es_hbm.at[pl.ds(row0, CB)])
        pltpu.sync_copy(buf_r[b], er_hbm.at[pl.ds(row0, CB)])

    start(0, 0)
    start(1, 1)

    def body(g, carry):
        for b in range(2):
            j = 2 * g + b
            finish(j, b)
            start((j + 2) % CH, b)
        return carry

    lax.fori_loop(0, CH // 2, body, 0)
    finish(CH - 1, (CH - 1) % 2)
    bl = CH % 2
    pltpu.make_async_copy(ts_hbm.at[pl.ds(0, CB)], buf_s[bl],
                          sem_s[bl]).wait()
    pltpu.make_async_copy(tr_hbm.at[pl.ds(0, CB)], buf_r[bl],
                          sem_r[bl]).wait()


def _scatter_body(eo_hbm, rs_hbm, p0_hbm, p1_hbm,
                  ridx, eb0, eb1, acc, sr0, sr1):
    eb = (eb0, eb1)
    sem_r = (sr0, sr1)
    c = lax.axis_index("c")
    t = lax.axis_index("s")
    wid = t * NC + c
    ebase = wid * ET

    def start_read(j, b):
        pltpu.async_copy(eo_hbm.at[pl.ds(ebase + j * CB, CB)], eb[b],
                         sem_r[b])

    def drain_read(b):
        pltpu.make_async_copy(eo_hbm.at[pl.ds(0, CB)], eb[b],
                              sem_r[b]).wait()

    pltpu.sync_copy(rs_hbm.at[wid], ridx)

    def zrow(i, carry):
        for k in range(W // 16):
            eb0[i, pl.ds(k * 16, 16)] = jnp.zeros((16,), jnp.float32)
        return carry
    lax.fori_loop(0, CB, zrow, 0)
    for q in range(RPT // CB):
        pltpu.sync_copy(eb0, acc.at[pl.ds(t * RPT + q * CB, CB)])
    start_read(0, 0)
    start_read(1, 1)
    plsc.subcore_barrier()

    def body(g, carry):
        for b in range(2):
            j = 2 * g + b
            drain_read(b)
            pltpu.sync_copy(eb[b], acc.at[ridx.at[j]], add=True)
            start_read((j + 2) % CH, b)
        return carry

    lax.fori_loop(0, CH // 2, body, 0)
    jt = CH - 1
    drain_read(jt % 2)
    pltpu.sync_copy(eb[jt % 2], acc.at[ridx.at[jt]], add=True)
    drain_read(CH % 2)
    plsc.subcore_barrier()

    for q in range(RPT // CB):
        row0 = t * RPT + q * CB
        pltpu.sync_copy(acc.at[pl.ds(row0, CB)], eb0)

        @pl.when(c == 0)
        def _():
            pltpu.sync_copy(eb0, p0_hbm.at[pl.ds(row0, CB)])

        @pl.when(c == 1)
        def _():
            pltpu.sync_copy(eb0, p1_hbm.at[pl.ds(row0, CB)])


@functools.lru_cache(maxsize=1)
def _sc_kernels():
    mesh = plsc.VectorSubcoreMesh(
        core_axis_name="c", subcore_axis_name="s",
        num_cores=NC, num_subcores=NS)
    gather = pl.kernel(
        _gather_body,
        out_type=(jax.ShapeDtypeStruct((EP, W), jnp.float32),
                  jax.ShapeDtypeStruct((EP, W), jnp.float32)),
        mesh=mesh,
        scratch_types=(
            [pltpu.VMEM((CH, CB), jnp.int32)] * 2
            + [pltpu.VMEM((CB, W), jnp.float32)] * 4
            + [pltpu.SemaphoreType.DMA] * 4
        ),
    )
    scatter = pl.kernel(
        _scatter_body,
        out_type=(jax.ShapeDtypeStruct((NP, W), jnp.float32),
                  jax.ShapeDtypeStruct((NP, W), jnp.float32)),
        mesh=mesh,
        scratch_types=(
            [pltpu.VMEM((CH, CB), jnp.int32)]
            + [pltpu.VMEM((CB, W), jnp.float32)] * 2
            + [pltpu.VMEM_SHARED((NP, W), jnp.float32)]
            + [pltpu.SemaphoreType.DMA] * 2
        ),
    )
    return gather, scatter


def _embed_body(x_ref, pos_ref, win_ref, bin_ref, h_ref, ts_ref, tr_ref):
    xb = x_ref[...]
    pb = pos_ref[...]
    h0 = xb @ win_ref[...] + bin_ref[...]
    z = jnp.zeros((xb.shape[0], W - H - 4), jnp.float32)
    h_ref[...] = h0
    ts_ref[...] = jnp.concatenate([h0, pb, z], axis=1)
    tr_ref[...] = jnp.concatenate([h0, -pb, z], axis=1)


def _edge_body(es_ref, er_ref, we1_ref, be1_ref, we2_ref, be2_ref,
               wp1_ref, bp1_ref, wp2_ref, bp2_ref, eo_ref):
    es = es_ref[...]
    er = er_ref[...]
    d16 = es[:, H:W] + er[:, H:W]
    dsq = d16 * d16
    radial = dsq[:, 0:1] + dsq[:, 1:2] + dsq[:, 2:3]
    cc = jnp.concatenate([es[:, :H], er[:, :H], radial], axis=1)
    m = _silu(cc @ we1_ref[...] + be1_ref[...])
    m = _silu(m @ we2_ref[...] + be2_ref[...])
    q1 = _silu(m @ wp1_ref[...] + bp1_ref[...])
    pw = q1 @ wp2_ref[...] + bp2_ref[...]
    t16 = jnp.clip(d16 * pw, -100.0, 100.0)
    eo_ref[...] = jnp.concatenate([m, t16], axis=1)


def _node_body(h_ref, pos_ref, p0_ref, p1_ref, wn1_ref, bn1_ref,
               wn2_ref, bn2_ref, hn_ref, posn_ref, ts_ref, tr_ref):
    h = h_ref[...]
    p = p0_ref[...] + p1_ref[...]
    u1 = _silu(jnp.concatenate([h, p[:, :H]], axis=1) @ wn1_ref[...]
               + bn1_ref[...])
    hn = h + (u1 @ wn2_ref[...] + bn2_ref[...])
    posn = pos_ref[...] + p[:, H:H + 4]
    z = jnp.zeros((h.shape[0], W - H - 4), jnp.float32)
    hn_ref[...] = hn
    posn_ref[...] = posn
    ts_ref[...] = jnp.concatenate([hn, posn, z], axis=1)
    tr_ref[...] = jnp.concatenate([hn, -posn, z], axis=1)


def _node_last_body(h_ref, pos_ref, p0_ref, p1_ref, wn1_ref, bn1_ref,
                    wn2_ref, bn2_ref, wout_ref, bout_ref,
                    out_ref, posn_ref):
    h = h_ref[...]
    p = p0_ref[...] + p1_ref[...]
    u1 = _silu(jnp.concatenate([h, p[:, :H]], axis=1) @ wn1_ref[...]
               + bn1_ref[...])
    hn = h + (u1 @ wn2_ref[...] + bn2_ref[...])
    out_ref[...] = hn @ wout_ref[...] + bout_ref[...]
    posn_ref[...] = pos_ref[...] + p[:, H:H + 4]


def _full(shape):
    return pl.BlockSpec(shape, lambda i: (0,) * len(shape))


def _rows(b, w):
    return pl.BlockSpec((b, w), lambda i: (i, 0))


def _tc_embed(xp, posp, w_in, b_in):
    return pl.pallas_call(
        _embed_body,
        grid=(NP // BN,),
        in_specs=[_rows(BN, D_IN), _rows(BN, 4), _full((D_IN, H)),
                  _full((1, H))],
        out_specs=[_rows(BN, H), _rows(BN, W), _rows(BN, W)],
        out_shape=[jax.ShapeDtypeStruct((NP, H), jnp.float32),
                   jax.ShapeDtypeStruct((NP, W), jnp.float32),
                   jax.ShapeDtypeStruct((NP, W), jnp.float32)],
    )(xp, posp, w_in, b_in)


def _tc_edge(es, er, we1, be1, we2, be2, wp1, bp1, wp2, bp2):
    return pl.pallas_call(
        _edge_body,
        grid=(EP // BE,),
        in_specs=[_rows(BE, W), _rows(BE, W), _full((2 * H + 1, H)),
                  _full((1, H)), _full((H, H)), _full((1, H)), _full((H, H)),
                  _full((1, H)), _full((H, 1)), _full((1, 1))],
        out_specs=[_rows(BE, W)],
        out_shape=[jax.ShapeDtypeStruct((EP, W), jnp.float32)],
    )(es, er, we1, be1, we2, be2, wp1, bp1, wp2, bp2)[0]


def _tc_node(h, posp, p0, p1, wn1, bn1, wn2, bn2):
    return pl.pallas_call(
        _node_body,
        grid=(NP // BN,),
        in_specs=[_rows(BN, H), _rows(BN, 4), _rows(BN, W), _rows(BN, W),
                  _full((2 * H, H)), _full((1, H)), _full((H, H)),
                  _full((1, H))],
        out_specs=[_rows(BN, H), _rows(BN, 4), _rows(BN, W), _rows(BN, W)],
        out_shape=[jax.ShapeDtypeStruct((NP, H), jnp.float32),
                   jax.ShapeDtypeStruct((NP, 4), jnp.float32),
                   jax.ShapeDtypeStruct((NP, W), jnp.float32),
                   jax.ShapeDtypeStruct((NP, W), jnp.float32)],
    )(h, posp, p0, p1, wn1, bn1, wn2, bn2)


def _tc_node_last(h, posp, p0, p1, wn1, bn1, wn2, bn2, w_out, b_out):
    return pl.pallas_call(
        _node_last_body,
        grid=(NP // BN,),
        in_specs=[_rows(BN, H), _rows(BN, 4), _rows(BN, W), _rows(BN, W),
                  _full((2 * H, H)), _full((1, H)), _full((H, H)),
                  _full((1, H)), _full((H, H)), _full((1, H))],
        out_specs=[_rows(BN, H), _rows(BN, 4)],
        out_shape=[jax.ShapeDtypeStruct((NP, H), jnp.float32),
                   jax.ShapeDtypeStruct((NP, 4), jnp.float32)],
    )(h, posp, p0, p1, wn1, bn1, wn2, bn2, w_out, b_out)


def kernel(x, pos, edge_index, W_in, b_in, We1, be1, We2, be2, Wp1, bp1,
           Wp2, bp2, Wn1, bn1, Wn2, bn2, W_out, b_out):
    f32 = jnp.float32
    s = edge_index[0].astype(jnp.int32)
    r = edge_index[1].astype(jnp.int32)
    padg = jnp.zeros((EP - E,), jnp.int32)
    sg = jnp.concatenate([s, padg]).reshape(NW, CH, CB)
    rg = jnp.concatenate([r, padg]).reshape(NW, CH, CB)
    rs = jnp.concatenate(
        [r, jnp.full((EP - E,), NP - 1, jnp.int32)]).reshape(NW, CH, CB)

    xp = jnp.zeros((NP, D_IN), f32).at[:N].set(x)
    posp = jnp.zeros((NP, 4), f32).at[:N, :3].set(pos)
    r2 = lambda v: v.reshape(1, -1)

    sc_gather, sc_scatter = _sc_kernels()
    h, ts, tr = _tc_embed(xp, posp, W_in, r2(b_in))
    for l in range(L):
        es, er = sc_gather(ts, tr, sg, rg)
        eo = _tc_edge(es, er, We1[l], r2(be1[l]), We2[l], r2(be2[l]),
                      Wp1[l], r2(bp1[l]), Wp2[l], r2(bp2[l]))
        p0, p1 = sc_scatter(eo, rs)
        if l < L - 1:
            h, posp, ts, tr = _tc_node(
                h, posp, p0, p1, Wn1[l], r2(bn1[l]), Wn2[l], r2(bn2[l]))
        else:
            out, posp = _tc_node_last(
                h, posp, p0, p1, Wn1[l], r2(bn1[l]), Wn2[l], r2(bn2[l]),
                W_out, r2(b_out))
    return (out[:N], posp[:N, :3])
